# baseline (speedup 1.0000x reference)
import jax
import jax.numpy as jnp
from jax import lax
from jax.experimental import pallas as pl
from jax.experimental.pallas import tpu as pltpu
from jax.experimental.pallas import tpu_sc as plsc

MAX_REL = 64
VOCAB = 2 * MAX_REL + 1
ROW = 128
N = 512
LANES = 16
NR = 8
SW = 1024


def _strips_body(table_hbm, strips_hbm, idx_v, buf_v, gsem):
    nc = plsc.get_sparse_core_info().num_cores
    wid = lax.axis_index("s") * nc + lax.axis_index("c")
    r = wid // 2
    h = wid % 2

    @pl.when(wid < 2 * NR)
    def _build():
        lane = lax.iota(jnp.int32, LANES)
        for c in range(4):
            for j in range(128 // LANES):
                m = lane + j * LANES + c * 128 + h * 512
                idx_v[pl.ds(j * LANES, LANES)] = (
                    jnp.clip(m - (N - NR) - r, -MAX_REL, MAX_REL) + MAX_REL)
            pltpu.async_copy(table_hbm.at[idx_v],
                             buf_v.at[pl.ds(c * 128, 128)], gsem).wait()
        pltpu.sync_copy(buf_v, strips_hbm.at[r, pl.ds(h * 512, 512)])


def _expand_body(strips_ref, eye_ref, out_ref):
    t = pl.program_id(1)
    off = pl.multiple_of(8 * (63 - t), 8)
    w = strips_ref[0, pl.ds(off, N), :]
    out_ref[0] = lax.dot_general(
        eye_ref[...], w, (((1,), (1,)), ((), ())),
        preferred_element_type=jnp.float32)


def kernel(len_in, len_out, table):
    del len_in, len_out
    mesh = plsc.VectorSubcoreMesh(core_axis_name="c", subcore_axis_name="s")
    build = pl.kernel(
        _strips_body,
        mesh=mesh,
        out_type=jax.ShapeDtypeStruct((NR, SW, ROW), jnp.float32),
        scratch_types=[
            pltpu.VMEM((128,), jnp.int32),
            pltpu.VMEM((512, ROW), jnp.float32),
            pltpu.SemaphoreType.DMA,
        ],
    )
    strips = build(table)

    out_phys = pl.pallas_call(
        _expand_body,
        grid=(NR, N // NR),
        in_specs=[pl.BlockSpec((1, SW, ROW), lambda r, t: (r, 0, 0)),
                  pl.BlockSpec((ROW, ROW), lambda r, t: (0, 0))],
        out_specs=pl.BlockSpec((1, ROW, N), lambda r, t: (NR * t + r, 0, 0)),
        out_shape=jax.ShapeDtypeStruct((N, ROW, N), jnp.float32),
    )(strips, jnp.eye(ROW, dtype=jnp.float32))

    return jnp.transpose(out_phys.reshape(N, 8, 16, N), (0, 3, 1, 2))


# speedup vs baseline: 1.0079x; 1.0079x over previous
import jax
import jax.numpy as jnp
from jax import lax
from jax.experimental import pallas as pl
from jax.experimental.pallas import tpu as pltpu
from jax.experimental.pallas import tpu_sc as plsc

MAX_REL = 64
VOCAB = 2 * MAX_REL + 1
ROW = 128
N = 512
LANES = 16
NR = 8
SW = 1024


def _strips_body(table_hbm, strips_hbm, idx_v, buf_v, gsem):
    nc = plsc.get_sparse_core_info().num_cores
    wid = lax.axis_index("s") * nc + lax.axis_index("c")
    r = wid // 2
    h = wid % 2

    @pl.when(wid < 2 * NR)
    def _build():
        lane = lax.iota(jnp.int32, LANES)
        for c in range(4):
            for j in range(128 // LANES):
                m = lane + j * LANES + c * 128 + h * 512
                idx_v[pl.ds(j * LANES, LANES)] = (
                    jnp.clip(m - (N - NR) - r, -MAX_REL, MAX_REL) + MAX_REL)
            pltpu.async_copy(table_hbm.at[idx_v],
                             buf_v.at[pl.ds(c * 128, 128)], gsem).wait()
        pltpu.sync_copy(buf_v, strips_hbm.at[r, pl.ds(h * 512, 512)])


def _expand_body(strips_ref, eye_ref, out_ref):
    t = pl.program_id(0)
    r = pl.program_id(1)
    off = pl.multiple_of(8 * (63 - t), 8)
    w = strips_ref[r, pl.ds(off, N), :]
    out_ref[0] = lax.dot_general(
        eye_ref[...], w, (((1,), (1,)), ((), ())),
        preferred_element_type=jnp.float32)


def kernel(len_in, len_out, table):
    del len_in, len_out
    mesh = plsc.VectorSubcoreMesh(core_axis_name="c", subcore_axis_name="s")
    build = pl.kernel(
        _strips_body,
        mesh=mesh,
        out_type=jax.ShapeDtypeStruct((NR, SW, ROW), jnp.float32),
        scratch_types=[
            pltpu.VMEM((128,), jnp.int32),
            pltpu.VMEM((512, ROW), jnp.float32),
            pltpu.SemaphoreType.DMA,
        ],
    )
    strips = build(table)

    out_phys = pl.pallas_call(
        _expand_body,
        grid=(N // NR, NR),
        in_specs=[pl.BlockSpec((NR, SW, ROW), lambda t, r: (0, 0, 0)),
                  pl.BlockSpec((ROW, ROW), lambda t, r: (0, 0))],
        out_specs=pl.BlockSpec((1, ROW, N), lambda t, r: (NR * t + r, 0, 0)),
        out_shape=jax.ShapeDtypeStruct((N, ROW, N), jnp.float32),
    )(strips, jnp.eye(ROW, dtype=jnp.float32))

    return jnp.transpose(out_phys.reshape(N, 8, 16, N), (0, 3, 1, 2))


# speedup vs baseline: 3.1121x; 3.0876x over previous
import jax
import jax.numpy as jnp
from jax import lax
from jax.experimental import pallas as pl
from jax.experimental.pallas import tpu as pltpu
from jax.experimental.pallas import tpu_sc as plsc

MAX_REL = 64
VOCAB = 2 * MAX_REL + 1
ROW = 128
N = 512
LANES = 16
NR = 8
SW = 1024
EXT = 1152


def _strips_body(table_hbm, strips_hbm, idx_v, buf_v, ext_sh, gsem):
    nc = plsc.get_sparse_core_info().num_cores
    ns = plsc.get_sparse_core_info().num_subcores
    sid = lax.axis_index("s")
    wid = sid * nc + lax.axis_index("c")

    @pl.when(sid < EXT // 128)
    def _build():
        lane = lax.iota(jnp.int32, LANES)
        for j in range(128 // LANES):
            m = lane + j * LANES + sid * 128
            idx_v[pl.ds(j * LANES, LANES)] = (
                jnp.clip(m - (N + NR - 1), -MAX_REL, MAX_REL) + MAX_REL)
        pltpu.async_copy(table_hbm.at[idx_v], buf_v, gsem).wait()
        pltpu.sync_copy(buf_v, ext_sh.at[pl.ds(sid * 128, 128)])
    plsc.subcore_barrier()

    r = sid // 2
    h = sid % 2
    pltpu.sync_copy(ext_sh.at[pl.ds(15 - r + h * 512, 512)],
                    strips_hbm.at[r, pl.ds(h * 512, 512)])


def _expand_body(strips_ref, eye_ref, out_ref):
    t = pl.program_id(0)
    off = pl.multiple_of(8 * (63 - t), 8)
    for r in range(NR):
        w = strips_ref[r, pl.ds(off, N), :]
        out_ref[r] = lax.dot_general(
            eye_ref[...], w, (((1,), (1,)), ((), ())),
            preferred_element_type=jnp.float32)


def kernel(len_in, len_out, table):
    del len_in, len_out
    mesh = plsc.VectorSubcoreMesh(core_axis_name="c", subcore_axis_name="s")
    build = pl.kernel(
        _strips_body,
        mesh=mesh,
        out_type=jax.ShapeDtypeStruct((NR, SW, ROW), jnp.float32),
        scratch_types=[
            pltpu.VMEM((128,), jnp.int32),
            pltpu.VMEM((128, ROW), jnp.float32),
            pltpu.VMEM_SHARED((EXT, ROW), jnp.float32),
            pltpu.SemaphoreType.DMA,
        ],
    )
    strips = build(table)

    out_phys = pl.pallas_call(
        _expand_body,
        grid=(N // NR,),
        in_specs=[pl.BlockSpec((NR, SW, ROW), lambda t: (0, 0, 0)),
                  pl.BlockSpec((ROW, ROW), lambda t: (0, 0))],
        out_specs=pl.BlockSpec((NR, ROW, N), lambda t: (t, 0, 0)),
        out_shape=jax.ShapeDtypeStruct((N, ROW, N), jnp.float32),
    )(strips, jnp.eye(ROW, dtype=jnp.float32))

    return jnp.transpose(out_phys.reshape(N, 8, 16, N), (0, 3, 1, 2))


# speedup vs baseline: 4.6005x; 1.4783x over previous
import jax
import jax.numpy as jnp
from jax import lax
from jax.experimental import pallas as pl
from jax.experimental.pallas import tpu as pltpu
from jax.experimental.pallas import tpu_sc as plsc

MAX_REL = 64
VOCAB = 2 * MAX_REL + 1
ROW = 128
N = 512
LANES = 16
NR = 8
SW = 1024
EXT = 1152


_CHUNKS = [
    (0, 128, 0), (128, 128, 0), (256, 128, 0), (384, 71, 0),
    (583, 128, 128), (711, 128, 128), (839, 128, 128), (967, 128, 128),
    (1095, 57, 128),
]


def _strips_body(table_hbm, strips_hbm, buf_v, ext_sh, gsem):
    nc = plsc.get_sparse_core_info().num_cores
    sid = lax.axis_index("s")

    @pl.when(sid == 0)
    def _mid():
        pltpu.async_copy(table_hbm.at[pl.ds(0, 128)], buf_v, gsem).wait()
        pltpu.sync_copy(buf_v, ext_sh.at[pl.ds(455, 128)])

    for b, (start, length, trow) in enumerate(_CHUNKS):
        @pl.when(sid == b + 1)
        def _fill(start=start, length=length, trow=trow):
            pltpu.async_copy(table_hbm.at[pl.ds(trow, 1)],
                             buf_v.at[pl.ds(0, 1)], gsem).wait()
            row = [buf_v[0, pl.ds(j * LANES, LANES)]
                   for j in range(ROW // LANES)]

            def _rep(m, carry):
                for j in range(ROW // LANES):
                    buf_v[m, pl.ds(j * LANES, LANES)] = row[j]
                return carry
            lax.fori_loop(1, length, _rep, 0)
            pltpu.sync_copy(buf_v.at[pl.ds(0, length)],
                            ext_sh.at[pl.ds(start, length)])
    plsc.subcore_barrier()

    r = sid // 2
    h = sid % 2
    pltpu.sync_copy(ext_sh.at[pl.ds(15 - r + h * 512, 512)],
                    strips_hbm.at[r, pl.ds(h * 512, 512)])


def _expand_body(strips_ref, eye_ref, out_ref):
    t = pl.program_id(0)
    off = pl.multiple_of(8 * (63 - t), 8)
    for r in range(NR):
        w = strips_ref[r, pl.ds(off, N), :]
        out_ref[r] = lax.dot_general(
            eye_ref[...], w, (((1,), (1,)), ((), ())),
            preferred_element_type=jnp.float32)


def kernel(len_in, len_out, table):
    del len_in, len_out
    mesh = plsc.VectorSubcoreMesh(core_axis_name="c", subcore_axis_name="s")
    build = pl.kernel(
        _strips_body,
        mesh=mesh,
        out_type=jax.ShapeDtypeStruct((NR, SW, ROW), jnp.float32),
        scratch_types=[
            pltpu.VMEM((128, ROW), jnp.float32),
            pltpu.VMEM_SHARED((EXT, ROW), jnp.float32),
            pltpu.SemaphoreType.DMA,
        ],
    )
    strips = build(table)

    out_phys = pl.pallas_call(
        _expand_body,
        grid=(N // NR,),
        in_specs=[pl.BlockSpec((NR, SW, ROW), lambda t: (0, 0, 0)),
                  pl.BlockSpec((ROW, ROW), lambda t: (0, 0))],
        out_specs=pl.BlockSpec((NR, ROW, N), lambda t: (t, 0, 0)),
        out_shape=jax.ShapeDtypeStruct((N, ROW, N), jnp.float32),
    )(strips, jnp.eye(ROW, dtype=jnp.float32))

    return jnp.transpose(out_phys.reshape(N, 8, 16, N), (0, 3, 1, 2))


# speedup vs baseline: 5.4006x; 1.1739x over previous
import jax
import jax.numpy as jnp
from jax import lax
from jax.experimental import pallas as pl
from jax.experimental.pallas import tpu as pltpu
from jax.experimental.pallas import tpu_sc as plsc

MAX_REL = 64
VOCAB = 2 * MAX_REL + 1
ROW = 128
N = 512
LANES = 16
NR = 8
SW = 1024
EXT = 1152


_CHUNKS = [
    (0, 128, 0), (128, 128, 0), (256, 128, 0), (384, 71, 0),
    (583, 128, 128), (711, 128, 128), (839, 128, 128), (967, 128, 128),
    (1095, 57, 128),
]


def _strips_body(table_hbm, strips_hbm, buf_v, ext_sh, gsem):
    nc = plsc.get_sparse_core_info().num_cores
    sid = lax.axis_index("s")

    @pl.when(sid == 0)
    def _mid():
        pltpu.async_copy(table_hbm.at[pl.ds(0, 128)], buf_v, gsem).wait()
        pltpu.sync_copy(buf_v, ext_sh.at[pl.ds(455, 128)])

    for b, (start, length, trow) in enumerate(_CHUNKS):
        @pl.when(sid == b + 1)
        def _fill(start=start, length=length, trow=trow):
            pltpu.async_copy(table_hbm.at[pl.ds(trow, 1)],
                             buf_v.at[pl.ds(0, 1)], gsem).wait()
            row = [buf_v[0, pl.ds(j * LANES, LANES)]
                   for j in range(ROW // LANES)]

            def _rep(m, carry):
                for j in range(ROW // LANES):
                    buf_v[m, pl.ds(j * LANES, LANES)] = row[j]
                return carry
            lax.fori_loop(1, length, _rep, 0)
            pltpu.sync_copy(buf_v.at[pl.ds(0, length)],
                            ext_sh.at[pl.ds(start, length)])
    plsc.subcore_barrier()

    r = sid // 2
    h = sid % 2
    pltpu.sync_copy(ext_sh.at[pl.ds(15 - r + h * 512, 512)],
                    strips_hbm.at[r, pl.ds(h * 512, 512)])


def _expand_body(strips_ref, eye_ref, out_ref):
    s = pl.program_id(0)
    for half in range(2):
        t = 2 * s + half
        off = pl.multiple_of(8 * (63 - t), 8)
        for r in range(NR):
            w = strips_ref[r, pl.ds(off, N), :]
            out_ref[half * NR + r] = lax.dot_general(
                eye_ref[...], w, (((1,), (1,)), ((), ())),
                preferred_element_type=jnp.float32)


def kernel(len_in, len_out, table):
    del len_in, len_out
    mesh = plsc.VectorSubcoreMesh(core_axis_name="c", subcore_axis_name="s")
    build = pl.kernel(
        _strips_body,
        mesh=mesh,
        out_type=jax.ShapeDtypeStruct((NR, SW, ROW), jnp.float32),
        scratch_types=[
            pltpu.VMEM((128, ROW), jnp.float32),
            pltpu.VMEM_SHARED((EXT, ROW), jnp.float32),
            pltpu.SemaphoreType.DMA,
        ],
    )
    strips = build(table)

    out_phys = pl.pallas_call(
        _expand_body,
        grid=(N // (2 * NR),),
        in_specs=[pl.BlockSpec((NR, SW, ROW), lambda s: (0, 0, 0)),
                  pl.BlockSpec((ROW, ROW), lambda s: (0, 0))],
        out_specs=pl.BlockSpec((2 * NR, ROW, N), lambda s: (s, 0, 0)),
        out_shape=jax.ShapeDtypeStruct((N, ROW, N), jnp.float32),
    )(strips, jnp.eye(ROW, dtype=jnp.float32))

    return jnp.transpose(out_phys.reshape(N, 8, 16, N), (0, 3, 1, 2))


# speedup vs baseline: 5.6037x; 1.0376x over previous
import jax
import jax.numpy as jnp
from jax import lax
from jax.experimental import pallas as pl
from jax.experimental.pallas import tpu as pltpu
from jax.experimental.pallas import tpu_sc as plsc

MAX_REL = 64
VOCAB = 2 * MAX_REL + 1
ROW = 128
N = 512
LANES = 16
NR = 8
SW = 1024
EXT = 1152


_CHUNKS = [
    (0, 128, 0), (128, 128, 0), (256, 128, 0), (384, 71, 0),
    (583, 128, 128), (711, 128, 128), (839, 128, 128), (967, 128, 128),
    (1095, 57, 128),
]


def _strips_body(table_hbm, strips_hbm, buf_v, ext_sh, gsem):
    nc = plsc.get_sparse_core_info().num_cores
    sid = lax.axis_index("s")

    @pl.when(sid == 0)
    def _mid():
        pltpu.async_copy(table_hbm.at[pl.ds(0, 128)], buf_v, gsem).wait()
        pltpu.sync_copy(buf_v, ext_sh.at[pl.ds(455, 128)])

    for b, (start, length, trow) in enumerate(_CHUNKS):
        @pl.when(sid == b + 1)
        def _fill(start=start, length=length, trow=trow):
            pltpu.async_copy(table_hbm.at[pl.ds(trow, 1)],
                             buf_v.at[pl.ds(0, 1)], gsem).wait()
            row = [buf_v[0, pl.ds(j * LANES, LANES)]
                   for j in range(ROW // LANES)]

            def _rep(m, carry):
                for j in range(ROW // LANES):
                    buf_v[m, pl.ds(j * LANES, LANES)] = row[j]
                return carry
            lax.fori_loop(1, length, _rep, 0)
            pltpu.sync_copy(buf_v.at[pl.ds(0, length)],
                            ext_sh.at[pl.ds(start, length)])
    plsc.subcore_barrier()

    r = sid // 2
    h = sid % 2
    pltpu.sync_copy(ext_sh.at[pl.ds(15 - r + h * 512, 512)],
                    strips_hbm.at[r, pl.ds(h * 512, 512)])


def _expand_body(strips_ref, eye_ref, out_ref):
    s = pl.program_id(0)
    for half in range(4):
        t = 4 * s + half
        off = pl.multiple_of(8 * (63 - t), 8)
        for r in range(NR):
            w = strips_ref[r, pl.ds(off, N), :]
            out_ref[half * NR + r] = lax.dot_general(
                eye_ref[...], w, (((1,), (1,)), ((), ())),
                preferred_element_type=jnp.float32)


def kernel(len_in, len_out, table):
    del len_in, len_out
    mesh = plsc.VectorSubcoreMesh(core_axis_name="c", subcore_axis_name="s")
    build = pl.kernel(
        _strips_body,
        mesh=mesh,
        out_type=jax.ShapeDtypeStruct((NR, SW, ROW), jnp.float32),
        scratch_types=[
            pltpu.VMEM((128, ROW), jnp.float32),
            pltpu.VMEM_SHARED((EXT, ROW), jnp.float32),
            pltpu.SemaphoreType.DMA,
        ],
    )
    strips = build(table)

    out_phys = pl.pallas_call(
        _expand_body,
        grid=(N // (4 * NR),),
        in_specs=[pl.BlockSpec((NR, SW, ROW), lambda s: (0, 0, 0)),
                  pl.BlockSpec((ROW, ROW), lambda s: (0, 0))],
        out_specs=pl.BlockSpec((4 * NR, ROW, N), lambda s: (s, 0, 0)),
        out_shape=jax.ShapeDtypeStruct((N, ROW, N), jnp.float32),
    )(strips, jnp.eye(ROW, dtype=jnp.float32))

    return jnp.transpose(out_phys.reshape(N, 8, 16, N), (0, 3, 1, 2))
